# Initial kernel scaffold; baseline (speedup 1.0000x reference)
import jax
import jax.numpy as jnp
from jax.experimental import pallas as pl


def kernel(points):
    raise NotImplementedError("write your pallas kernel here")


# speedup vs baseline: 9.0251x; 9.0251x over previous
import jax
import jax.numpy as jnp
from jax import lax
from jax.experimental import pallas as pl
from jax.experimental.pallas import tpu as pltpu
from jax.experimental.pallas import tpu_sc as plsc

_B, _C, _N = 8, 3, 16384
_K = 2048
_L = 16
_TPB = 4
_CHUNK = _N // _TPB
_NV = _CHUNK // _L
_W = 2 * _L


def _fps_body(pts, out, x, y, z, dists, outv, pair, buf, shared):
    c = lax.axis_index("c")
    s = lax.axis_index("s")
    batch = c * (_B // 2) + s // _TPB
    t = s % _TPB
    base = t * _CHUNK

    pltpu.sync_copy(pts.at[batch, 0], x)
    pltpu.sync_copy(pts.at[batch, 1], y)
    pltpu.sync_copy(pts.at[batch, 2], z)

    inf16 = jnp.full((_L,), jnp.inf, jnp.float32)

    def fill(i, _):
        dists[pl.ds(i * _L, _L)] = inf16
        return 0

    lax.fori_loop(0, _NV, fill, 0)

    lane = lax.iota(jnp.int32, _L)
    lane0 = lane == 0
    zero16 = jnp.zeros((_L,), jnp.int32)

    def coords_of(widx):
        return (plsc.load_gather(x, [widx]),
                plsc.load_gather(y, [widx]),
                plsc.load_gather(z, [widx]))

    def record(kidx, lx, ly, lz):
        plsc.store_scatter(outv, [zero16, kidx], lx, mask=lane0)
        plsc.store_scatter(outv, [zero16 + 1, kidx], ly, mask=lane0)
        plsc.store_scatter(outv, [zero16 + 2, kidx], lz, mask=lane0)

    x0 = jnp.full((_L,), x[pl.ds(0, _L)][0])
    y0 = jnp.full((_L,), y[pl.ds(0, _L)][0])
    z0 = jnp.full((_L,), z[pl.ds(0, _L)][0])

    @pl.when(t == 0)
    def _():
        record(zero16, x0, y0, z0)

    def step(k, carry):
        lxv, lyv, lzv = carry

        def inner(i, car):
            mv, mi, iv = car
            sl = pl.ds(i * _L, _L)
            gsl = pl.ds(base + i * _L, _L)
            dx = x[gsl] - lxv
            dy = y[gsl] - lyv
            dz = z[gsl] - lzv
            d = (dx * dx + dy * dy) + dz * dz
            nd = jnp.minimum(dists[sl], d)
            dists[sl] = nd
            gt = nd > mv
            mv = jnp.where(gt, nd, mv)
            mi = jnp.where(gt, iv, mi)
            return mv, mi, iv + _L

        mv0 = jnp.full((_L,), -1.0, jnp.float32)
        mi0 = jnp.zeros((_L,), jnp.int32)
        iv0 = jnp.full((_L,), base, jnp.int32) + lane
        mv, mi, _ = lax.fori_loop(0, _NV, inner, (mv0, mi0, iv0), unroll=4)

        pair[pl.ds(0, _L)] = mv
        pair[pl.ds(_L, _L)] = mi.astype(jnp.float32)
        pltpu.sync_copy(pair, shared.at[pl.ds(s * _W, _W)])
        plsc.subcore_barrier()
        g0 = (s // _TPB) * _TPB
        pltpu.sync_copy(shared.at[pl.ds(g0 * _W, _TPB * _W)], buf)
        plsc.subcore_barrier()

        av = buf[pl.ds(0, _L)]
        ai = buf[pl.ds(_L, _L)].astype(jnp.int32)
        for tt in range(1, _TPB):
            bv = buf[pl.ds(tt * _W, _L)]
            bi = buf[pl.ds(tt * _W + _L, _L)].astype(jnp.int32)
            g = bv > av
            av = jnp.where(g, bv, av)
            ai = jnp.where(g, bi, ai)
        mx = jnp.max(av)
        cand = jnp.where(av == mx, ai, jnp.int32(1 << 30))
        win = jnp.min(cand)
        winv = jnp.full((_L,), win, jnp.int32)

        nlx, nly, nlz = coords_of(winv)

        @pl.when(t == 0)
        def _():
            record(jnp.full((_L,), k, jnp.int32), nlx, nly, nlz)

        return nlx, nly, nlz

    lax.fori_loop(1, _K, step, (x0, y0, z0))

    @pl.when(t == 0)
    def _():
        pltpu.sync_copy(outv, out.at[batch])


@jax.jit
def kernel(points):
    mesh = plsc.VectorSubcoreMesh(core_axis_name="c", subcore_axis_name="s")
    f = pl.kernel(
        _fps_body,
        out_type=jax.ShapeDtypeStruct((_B, _C, _K), jnp.float32),
        mesh=mesh,
        compiler_params=pltpu.CompilerParams(
            use_tc_tiling_on_sc=False, needs_layout_passes=False),
        scratch_types=[
            pltpu.VMEM((_N,), jnp.float32),
            pltpu.VMEM((_N,), jnp.float32),
            pltpu.VMEM((_N,), jnp.float32),
            pltpu.VMEM((_CHUNK,), jnp.float32),
            pltpu.VMEM((_C, _K), jnp.float32),
            pltpu.VMEM((_W,), jnp.float32),
            pltpu.VMEM((_TPB * _W,), jnp.float32),
            pltpu.VMEM_SHARED((16 * _W,), jnp.float32),
        ],
    )
    return f(points)


# speedup vs baseline: 9.1637x; 1.0154x over previous
import jax
import jax.numpy as jnp
from jax import lax
from jax.experimental import pallas as pl
from jax.experimental.pallas import tpu as pltpu
from jax.experimental.pallas import tpu_sc as plsc

_B, _C, _N = 8, 3, 16384
_K = 2048
_L = 16
_TPB = 4
_CHUNK = _N // _TPB
_NV = _CHUNK // _L
_W = 2 * _L


def _fps_body(pts, out, x, y, z, dists, outv, pair, buf, shared):
    c = lax.axis_index("c")
    s = lax.axis_index("s")
    batch = c * (_B // 2) + s // _TPB
    t = s % _TPB
    base = t * _CHUNK

    pltpu.sync_copy(pts.at[batch, 0], x)
    pltpu.sync_copy(pts.at[batch, 1], y)
    pltpu.sync_copy(pts.at[batch, 2], z)

    inf16 = jnp.full((_L,), jnp.inf, jnp.float32)

    def fill(i, _):
        dists[pl.ds(i * _L, _L)] = inf16
        return 0

    lax.fori_loop(0, _NV, fill, 0)

    lane = lax.iota(jnp.int32, _L)
    lane0 = lane == 0
    zero16 = jnp.zeros((_L,), jnp.int32)

    def coords_of(widx):
        return (plsc.load_gather(x, [widx]),
                plsc.load_gather(y, [widx]),
                plsc.load_gather(z, [widx]))

    def record(kidx, lx, ly, lz):
        plsc.store_scatter(outv, [zero16, kidx], lx, mask=lane0)
        plsc.store_scatter(outv, [zero16 + 1, kidx], ly, mask=lane0)
        plsc.store_scatter(outv, [zero16 + 2, kidx], lz, mask=lane0)

    x0 = jnp.full((_L,), x[pl.ds(0, _L)][0])
    y0 = jnp.full((_L,), y[pl.ds(0, _L)][0])
    z0 = jnp.full((_L,), z[pl.ds(0, _L)][0])

    @pl.when(t == 0)
    def _():
        record(zero16, x0, y0, z0)

    def step(k, carry):
        lxv, lyv, lzv = carry

        def inner(i, car):
            mv, mi, iv = car
            sl = pl.ds(i * _L, _L)
            gsl = pl.ds(base + i * _L, _L)
            dx = x[gsl] - lxv
            dy = y[gsl] - lyv
            dz = z[gsl] - lzv
            d = (dx * dx + dy * dy) + dz * dz
            nd = jnp.minimum(dists[sl], d)
            dists[sl] = nd
            gt = nd > mv
            mv = jnp.where(gt, nd, mv)
            mi = jnp.where(gt, iv, mi)
            return mv, mi, iv + _L

        mv0 = jnp.full((_L,), -1.0, jnp.float32)
        mi0 = jnp.zeros((_L,), jnp.int32)
        iv0 = jnp.full((_L,), base, jnp.int32) + lane
        mv, mi, _ = lax.fori_loop(0, _NV, inner, (mv0, mi0, iv0), unroll=8)

        pair[pl.ds(0, _L)] = mv
        pair[pl.ds(_L, _L)] = mi.astype(jnp.float32)
        p = lax.rem(k, 2) * (16 * _W)
        pltpu.sync_copy(pair, shared.at[pl.ds(p + s * _W, _W)])
        plsc.subcore_barrier()
        g0 = (s // _TPB) * _TPB
        pltpu.sync_copy(shared.at[pl.ds(p + g0 * _W, _TPB * _W)], buf)

        av = buf[pl.ds(0, _L)]
        ai = buf[pl.ds(_L, _L)].astype(jnp.int32)
        for tt in range(1, _TPB):
            bv = buf[pl.ds(tt * _W, _L)]
            bi = buf[pl.ds(tt * _W + _L, _L)].astype(jnp.int32)
            g = bv > av
            av = jnp.where(g, bv, av)
            ai = jnp.where(g, bi, ai)
        mx = jnp.max(av)
        cand = jnp.where(av == mx, ai, jnp.int32(1 << 30))
        win = jnp.min(cand)
        winv = jnp.full((_L,), win, jnp.int32)

        nlx, nly, nlz = coords_of(winv)

        @pl.when(t == 0)
        def _():
            record(jnp.full((_L,), k, jnp.int32), nlx, nly, nlz)

        return nlx, nly, nlz

    lax.fori_loop(1, _K, step, (x0, y0, z0))

    @pl.when(t == 0)
    def _():
        pltpu.sync_copy(outv, out.at[batch])


@jax.jit
def kernel(points):
    mesh = plsc.VectorSubcoreMesh(core_axis_name="c", subcore_axis_name="s")
    f = pl.kernel(
        _fps_body,
        out_type=jax.ShapeDtypeStruct((_B, _C, _K), jnp.float32),
        mesh=mesh,
        compiler_params=pltpu.CompilerParams(
            use_tc_tiling_on_sc=False, needs_layout_passes=False),
        scratch_types=[
            pltpu.VMEM((_N,), jnp.float32),
            pltpu.VMEM((_N,), jnp.float32),
            pltpu.VMEM((_N,), jnp.float32),
            pltpu.VMEM((_CHUNK,), jnp.float32),
            pltpu.VMEM((_C, _K), jnp.float32),
            pltpu.VMEM((_W,), jnp.float32),
            pltpu.VMEM((_TPB * _W,), jnp.float32),
            pltpu.VMEM_SHARED((2 * 16 * _W,), jnp.float32),
        ],
    )
    return f(points)


# speedup vs baseline: 26.9563x; 2.9416x over previous
import jax
import jax.numpy as jnp
from jax import lax
from jax.experimental import pallas as pl
from jax.experimental.pallas import tpu as pltpu
from jax.experimental.pallas import tpu_sc as plsc

_B, _C, _N = 8, 3, 16384
_K = 2048
_L = 16
_TPB = 4
_CHUNK = _N // _TPB
_NV = _CHUNK // _L
_W = 2 * _L


def _fps_body(pts, out, x, y, z, dists, outv, pair, buf, shared):
    c = lax.axis_index("c")
    s = lax.axis_index("s")
    batch = c * (_B // 2) + s // _TPB
    t = s % _TPB
    base = t * _CHUNK

    pltpu.sync_copy(pts.at[batch, 0], x)
    pltpu.sync_copy(pts.at[batch, 1], y)
    pltpu.sync_copy(pts.at[batch, 2], z)

    inf16 = jnp.full((_L,), jnp.inf, jnp.float32)

    def fill(i, _):
        dists[pl.ds(i * _L, _L)] = inf16
        return 0

    lax.fori_loop(0, _NV, fill, 0)

    lane = lax.iota(jnp.int32, _L)
    lane0 = lane == 0
    zero16 = jnp.zeros((_L,), jnp.int32)

    def coords_of(widx):
        return (plsc.load_gather(x, [widx]),
                plsc.load_gather(y, [widx]),
                plsc.load_gather(z, [widx]))

    def record(kidx, lx, ly, lz):
        plsc.store_scatter(outv, [zero16, kidx], lx, mask=lane0)
        plsc.store_scatter(outv, [zero16 + 1, kidx], ly, mask=lane0)
        plsc.store_scatter(outv, [zero16 + 2, kidx], lz, mask=lane0)

    x0 = jnp.full((_L,), x[pl.ds(0, _L)][0])
    y0 = jnp.full((_L,), y[pl.ds(0, _L)][0])
    z0 = jnp.full((_L,), z[pl.ds(0, _L)][0])

    @pl.when(t == 0)
    def _():
        record(zero16, x0, y0, z0)

    def step(k, carry):
        lxv, lyv, lzv = carry

        mv0 = jnp.full((_L,), -1.0, jnp.float32)
        mi0 = jnp.zeros((_L,), jnp.int32)
        iv0 = jnp.full((_L,), base, jnp.int32) + lane

        @plsc.parallel_loop(0, _NV, 1, unroll=8, carry=(mv0, mi0, iv0))
        def inner(i, car):
            mv, mi, iv = car
            sl = pl.ds(i * _L, _L)
            gsl = pl.ds(base + i * _L, _L)
            dx = x[gsl] - lxv
            dy = y[gsl] - lyv
            dz = z[gsl] - lzv
            d = (dx * dx + dy * dy) + dz * dz
            nd = jnp.minimum(dists[sl], d)
            dists[sl] = nd
            gt = nd > mv
            mv = jnp.where(gt, nd, mv)
            mi = jnp.where(gt, iv, mi)
            return mv, mi, iv + _L

        mv, mi, _ = inner

        pair[pl.ds(0, _L)] = mv
        pair[pl.ds(_L, _L)] = mi.astype(jnp.float32)
        p = lax.rem(k, 2) * (16 * _W)
        pltpu.sync_copy(pair, shared.at[pl.ds(p + s * _W, _W)])
        plsc.subcore_barrier()
        g0 = (s // _TPB) * _TPB
        pltpu.sync_copy(shared.at[pl.ds(p + g0 * _W, _TPB * _W)], buf)

        av = buf[pl.ds(0, _L)]
        ai = buf[pl.ds(_L, _L)].astype(jnp.int32)
        for tt in range(1, _TPB):
            bv = buf[pl.ds(tt * _W, _L)]
            bi = buf[pl.ds(tt * _W + _L, _L)].astype(jnp.int32)
            g = bv > av
            av = jnp.where(g, bv, av)
            ai = jnp.where(g, bi, ai)
        mx = jnp.max(av)
        cand = jnp.where(av == mx, ai, jnp.int32(1 << 30))
        win = jnp.min(cand)
        winv = jnp.full((_L,), win, jnp.int32)

        nlx, nly, nlz = coords_of(winv)

        @pl.when(t == 0)
        def _():
            record(jnp.full((_L,), k, jnp.int32), nlx, nly, nlz)

        return nlx, nly, nlz

    lax.fori_loop(1, _K, step, (x0, y0, z0))

    @pl.when(t == 0)
    def _():
        pltpu.sync_copy(outv, out.at[batch])


@jax.jit
def kernel(points):
    mesh = plsc.VectorSubcoreMesh(core_axis_name="c", subcore_axis_name="s")
    f = pl.kernel(
        _fps_body,
        out_type=jax.ShapeDtypeStruct((_B, _C, _K), jnp.float32),
        mesh=mesh,
        compiler_params=pltpu.CompilerParams(
            use_tc_tiling_on_sc=False, needs_layout_passes=False),
        scratch_types=[
            pltpu.VMEM((_N,), jnp.float32),
            pltpu.VMEM((_N,), jnp.float32),
            pltpu.VMEM((_N,), jnp.float32),
            pltpu.VMEM((_CHUNK,), jnp.float32),
            pltpu.VMEM((_C, _K), jnp.float32),
            pltpu.VMEM((_W,), jnp.float32),
            pltpu.VMEM((_TPB * _W,), jnp.float32),
            pltpu.VMEM_SHARED((2 * 16 * _W,), jnp.float32),
        ],
    )
    return f(points)
